# baseline (speedup 1.0000x reference)
import jax
import jax.numpy as jnp
from jax import lax
from jax.experimental import pallas as pl
from jax.experimental.pallas import tpu as pltpu
from jax.experimental.pallas import tpu_sc as plsc

_info = plsc.get_sparse_core_info()
_NC, _NS, _L = _info.num_cores, _info.num_subcores, _info.num_lanes
_NW = _NC * _NS
_B = 16384
_V = 1000000
_D = 64
_BPW = _B // _NW
_CHUNK = 128
_DBLK = 8
_NG = _BPW // _L


def _glove_body(cw_hbm, tw_hbm, cooc_hbm, wt_hbm, embv_hbm, embu_hbm,
                vbt_hbm,
                out_hbm,
                cw_v, tw_v, cooc_v, wt_v, cb_v, tb_v, cembT, tembT,
                idxc, idxt, acc_v, sem):
    wid = lax.axis_index("s") * _NC + lax.axis_index("c")
    base = pl.multiple_of(wid * _BPW, _BPW)

    pltpu.sync_copy(cw_hbm.at[pl.ds(base, _BPW)], cw_v)
    pltpu.sync_copy(tw_hbm.at[pl.ds(base, _BPW)], tw_v)
    pltpu.sync_copy(cooc_hbm.at[pl.ds(base, _BPW)], cooc_v)
    pltpu.sync_copy(wt_hbm.at[pl.ds(base, _BPW)], wt_v)

    vb1 = vbt_hbm.at[0]
    ev1 = embv_hbm.at[0]
    eu1 = embu_hbm.at[0]

    bias_copies = []
    for c in range(_BPW // _CHUNK):
        s = pl.ds(c * _CHUNK, _CHUNK)
        bias_copies.append(pltpu.async_copy(vb1.at[cw_v.at[s]], cb_v.at[s], sem))
        bias_copies.append(pltpu.async_copy(vb1.at[tw_v.at[s]], tb_v.at[s], sem))

    for blk in range(_D // _DBLK):
        copies = []
        for dd in range(_DBLK):
            d = blk * _DBLK + dd
            for c in range(_BPW // _L):
                s = pl.ds(c * _L, _L)
                idxc[dd, s] = cw_v[s] + (d * _V)
                idxt[dd, s] = tw_v[s] + (d * _V)
            for c in range(_BPW // _CHUNK):
                s = pl.ds(c * _CHUNK, _CHUNK)
                so = pl.ds(c * _CHUNK, _CHUNK)
                copies.append(pltpu.async_copy(
                    ev1.at[idxc.at[dd].at[s]], cembT.at[d].at[so], sem))
                copies.append(pltpu.async_copy(
                    eu1.at[idxt.at[dd].at[s]], tembT.at[d].at[so], sem))
        for cp in copies:
            cp.wait()
    for cp in bias_copies:
        cp.wait()

    acc = jnp.zeros((_L,), jnp.float32)

    def group(g, acc):
        e0 = pl.multiple_of(g * _L, _L)
        s = pl.ds(e0, _L)
        dot = cembT[0, s] * tembT[0, s]
        for d in range(1, _D):
            dot = dot + cembT[d, s] * tembT[d, s]
        err = dot + cb_v[s] + tb_v[s] - cooc_v[s]
        return acc + wt_v[s] * err * err

    acc = lax.fori_loop(0, _NG, group, acc)
    acc_v[...] = acc
    pltpu.sync_copy(acc_v, out_hbm.at[wid])


_glove_partials = pl.kernel(
    _glove_body,
    out_type=jax.ShapeDtypeStruct((_NW, _L), jnp.float32),
    mesh=plsc.VectorSubcoreMesh(core_axis_name="c", subcore_axis_name="s"),
    compiler_params=pltpu.CompilerParams(use_tc_tiling_on_sc=False),
    scratch_types=[
        pltpu.VMEM((_BPW,), jnp.int32),
        pltpu.VMEM((_BPW,), jnp.int32),
        pltpu.VMEM((_BPW,), jnp.float32),
        pltpu.VMEM((_BPW,), jnp.float32),
        pltpu.VMEM((_BPW,), jnp.float32),
        pltpu.VMEM((_BPW,), jnp.float32),
        pltpu.VMEM((_D, _BPW), jnp.float32),
        pltpu.VMEM((_D, _BPW), jnp.float32),
        pltpu.VMEM((_DBLK, _BPW), jnp.int32),
        pltpu.VMEM((_DBLK, _BPW), jnp.int32),
        pltpu.VMEM((_L,), jnp.float32),
        pltpu.SemaphoreType.DMA,
    ],
)


def _sum_body(x_ref, o_ref):
    o_ref[...] = jnp.sum(x_ref[...], keepdims=True)


def kernel(center_words, target_words, coocs, weights, emb_v, emb_u, v_bias,
           u_bias):
    del u_bias
    cw = center_words.reshape(_B)
    tw = target_words.reshape(_B)
    cooc = coocs.reshape(_B)
    wt = weights.reshape(_B)
    partials = _glove_partials(cw, tw, cooc, wt,
                               emb_v.T.reshape(1, _V * _D),
                               emb_u.T.reshape(1, _V * _D),
                               v_bias.T)
    total = pl.pallas_call(
        _sum_body,
        out_shape=jax.ShapeDtypeStruct((1, 1), jnp.float32),
    )(partials)
    return total[0, 0]
